# baseline (speedup 1.0000x reference)
import functools
import math

import jax
import jax.numpy as jnp
import numpy as np
from jax import lax
from jax.experimental import pallas as pl
from jax.experimental.pallas import tpu as pltpu

_SMOOTHING = 0.1
_BN = 32
_NBUF = 4


def _loss_body(eps, coef_g, c_row, n, v, tgt_sref, tgt_ref, pred_hbm,
               out_ref, buf, sems):
    nsteps = n // _BN
    lane = jax.lax.broadcasted_iota(jnp.int32, (1, 128), 1)

    def _start(j, slot):
        pltpu.make_async_copy(
            pred_hbm.at[pl.ds(j * _BN, _BN), :], buf.at[slot], sems.at[slot]
        ).start()

    for j in range(_NBUF):
        _start(j, j)

    def _slab(j, slot, acc):
        pltpu.make_async_copy(
            pred_hbm.at[pl.ds(j * _BN, _BN), :], buf.at[slot], sems.at[slot]
        ).wait()
        t = tgt_ref[pl.ds(j * _BN, _BN), :]
        x = buf[slot]
        valid = t != 0
        s = jnp.sum(x, axis=1, keepdims=True) - x[:, 0:1]
        part = jnp.sum(jnp.where(valid, s, 0.0))
        cnt = jnp.sum(jnp.where(valid, 1.0, 0.0))

        gpart = jnp.float32(0.0)
        for r in range(_BN):
            tr = tgt_sref[j * _BN + r]
            start = pl.multiple_of((tr // 128) * 128, 128)
            w = buf[slot, pl.ds(r, 1), pl.ds(start, 128)]
            gval = jnp.sum(jnp.where(lane == tr % 128, w, 0.0))
            gpart += jnp.where(tr != 0, gval, 0.0)

        @pl.when(j + _NBUF < nsteps)
        def _():
            _start(j + _NBUF, slot)

        return acc + (c_row * cnt - eps * part - coef_g * gpart)

    def _round(k, acc):
        j = k * _NBUF
        for slot in range(_NBUF):
            acc = _slab(j + slot, slot, acc)
        return acc

    acc = lax.fori_loop(0, nsteps // _NBUF, _round, jnp.float32(0.0))
    out_ref[0, 0] = acc


def kernel(pred, target):
    n, v = pred.shape
    eps = _SMOOTHING / (v - 2)
    eps32 = float(np.float32(eps))
    c_row = (v - 2) * (eps32 * math.log(eps32)) + 0.9 * math.log(0.9)
    coef_g = (1.0 - _SMOOTHING) - eps

    tgt2d = target.reshape(n, 1)
    grid_spec = pltpu.PrefetchScalarGridSpec(
        num_scalar_prefetch=1,
        grid=(1,),
        in_specs=[
            pl.BlockSpec((n, 1), lambda i, *_: (0, 0)),
            pl.BlockSpec(memory_space=pl.ANY),
        ],
        out_specs=pl.BlockSpec(
            (1, 1), lambda i, *_: (0, 0), memory_space=pltpu.SMEM
        ),
        scratch_shapes=[
            pltpu.VMEM((_NBUF, _BN, v), jnp.float32),
            pltpu.SemaphoreType.DMA((_NBUF,)),
        ],
    )
    out = pl.pallas_call(
        functools.partial(_loss_body, eps, coef_g, c_row, n, v),
        grid_spec=grid_spec,
        out_shape=jax.ShapeDtypeStruct((1, 1), jnp.float32),
    )(target, tgt2d, pred)
    return out[0, 0]
